# baseline (speedup 1.0000x reference)
import functools

import jax
import jax.numpy as jnp
from jax.experimental import pallas as pl

_L = 4
_K = 1024
_DZ = 64
_NC = 8
_SUB = 512
_TILE = _NC * _SUB


def _proj_in(x, win_ref, bin_ref):
    z = jnp.dot(x, win_ref[...], preferred_element_type=jnp.float32)
    return z + bin_ref[...]


def _scores(r, cbt, cn):
    e2 = jnp.dot((r + r).astype(jnp.bfloat16), cbt,
                 preferred_element_type=jnp.float32)
    rr = jnp.sum(r * r, axis=1, keepdims=True)
    return (rr - e2) + cn


def _first_hit(s, m, iota):
    idx = jnp.min(jnp.where(s == m, iota, _K), axis=1, keepdims=True)
    return (iota == idx).astype(jnp.float32).astype(jnp.bfloat16)


def _vq_body(x_ref, win_ref, bin_ref, cbp_ref, cbt_ref, cn_ref, wout_ref,
             bout_ref, out_ref):
    iota = jax.lax.broadcasted_iota(jnp.int32, (_SUB, _K), 1)
    z = [_proj_in(x_ref[c * _SUB:(c + 1) * _SUB], win_ref, bin_ref)
         for c in range(_NC)]
    r = list(z)
    qt = [jnp.zeros_like(z[0]) for _ in range(_NC)]
    for l in range(_L):
        cbt, cn, cbp = cbt_ref[l], cn_ref[l], cbp_ref[l]
        s = [_scores(r[c], cbt, cn) for c in range(_NC)]
        m = [jnp.min(s[c], axis=1, keepdims=True) for c in range(_NC)]
        mh = [(s[c] == m[c]).astype(jnp.float32).astype(jnp.bfloat16)
              for c in range(_NC)]
        q4 = [jnp.dot(mh[c], cbp, preferred_element_type=jnp.float32)
              for c in range(_NC)]
        nmax = q4[0][:, 3 * _DZ:3 * _DZ + 1]
        for c in range(1, _NC):
            nmax = jnp.maximum(nmax, q4[c][:, 3 * _DZ:3 * _DZ + 1])
        tied = jnp.max(nmax) > 1.5

        def _fix(s=s, m=m, cbp=cbp):
            return tuple(
                jnp.dot(_first_hit(s[c], m[c], iota), cbp,
                        preferred_element_type=jnp.float32)
                for c in range(_NC))

        q4 = jax.lax.cond(tied, _fix, lambda q4=q4: tuple(q4))
        for c in range(_NC):
            q = (q4[c][:, :_DZ] + q4[c][:, _DZ:2 * _DZ]) + \
                q4[c][:, 2 * _DZ:3 * _DZ]
            qt[c] = qt[c] + (r[c] + (q - r[c]))
            r[c] = r[c] - q
    for c in range(_NC):
        o = jnp.dot(qt[c].astype(jnp.bfloat16), wout_ref[...],
                    preferred_element_type=jnp.float32)
        out_ref[c * _SUB:(c + 1) * _SUB] = o + bout_ref[...]


def _trunc16(v):
    bits = jax.lax.bitcast_convert_type(v, jnp.uint32)
    return jax.lax.bitcast_convert_type(bits & jnp.uint32(0xFFFF0000),
                                        jnp.float32)


@functools.partial(jax.jit, static_argnames=("interpret",))
def kernel(x, proj_in_w, proj_in_b, codebooks, proj_out_w, proj_out_b,
           interpret=False):
    b, n, dims = x.shape
    tokens = b * n
    xf = x.reshape(tokens, dims).astype(jnp.bfloat16)
    cbt = jnp.swapaxes(codebooks, 1, 2).astype(jnp.bfloat16)
    hi = _trunc16(codebooks)
    rem = codebooks - hi
    mid = _trunc16(rem)
    lo = rem - mid
    ones = jnp.ones((_L, _K, 1), jnp.bfloat16)
    cbp = jnp.concatenate([hi.astype(jnp.bfloat16), mid.astype(jnp.bfloat16),
                           lo.astype(jnp.bfloat16), ones,
                           jnp.zeros((_L, _K, _DZ - 1), jnp.bfloat16)],
                          axis=-1)
    cn = jnp.sum(codebooks * codebooks, axis=-1)
    cn = cn.reshape(_L, 1, _K)
    win = proj_in_w.astype(jnp.bfloat16)
    wout = proj_out_w.astype(jnp.bfloat16)
    bin2 = proj_in_b.reshape(1, -1)
    bout2 = proj_out_b.reshape(1, -1)
    grid = (tokens // _TILE,)
    out = pl.pallas_call(
        _vq_body,
        grid=grid,
        in_specs=[
            pl.BlockSpec((_TILE, dims), lambda i: (i, 0)),
            pl.BlockSpec((dims, _DZ), lambda i: (0, 0)),
            pl.BlockSpec((1, _DZ), lambda i: (0, 0)),
            pl.BlockSpec((_L, _K, 4 * _DZ), lambda i: (0, 0, 0)),
            pl.BlockSpec((_L, _DZ, _K), lambda i: (0, 0, 0)),
            pl.BlockSpec((_L, 1, _K), lambda i: (0, 0, 0)),
            pl.BlockSpec((_DZ, dims), lambda i: (0, 0)),
            pl.BlockSpec((1, dims), lambda i: (0, 0)),
        ],
        out_specs=pl.BlockSpec((_TILE, dims), lambda i: (i, 0)),
        out_shape=jax.ShapeDtypeStruct((tokens, dims), jnp.float32),
        interpret=interpret,
    )(xf, win, bin2, cbp, cbt, cn, wout, bout2)
    return out.reshape(b, n, dims)


# speedup vs baseline: 1.3906x; 1.3906x over previous
import functools

import jax
import jax.numpy as jnp
from jax.experimental import pallas as pl

_L = 4
_K = 1024
_DZ = 64
_NC = 4
_SUB = 512
_TILE = _NC * _SUB


def _proj_in(x, win_ref, bin_ref):
    z = jnp.dot(x, win_ref[...], preferred_element_type=jnp.float32)
    return z + bin_ref[...]


def _scores(r, cbt, cn):
    e2 = jnp.dot((r + r).astype(jnp.bfloat16), cbt,
                 preferred_element_type=jnp.float32)
    rr = jnp.sum(r * r, axis=1, keepdims=True)
    return (rr - e2) + cn


def _first_hit(s, m, iota):
    idx = jnp.min(jnp.where(s == m, iota, _K), axis=1, keepdims=True)
    return (iota == idx).astype(jnp.float32).astype(jnp.bfloat16)


def _vq_body(x_ref, win_ref, bin_ref, cbp_ref, cbt_ref, cn_ref, wout_ref,
             bout_ref, out_ref):
    iota = jax.lax.broadcasted_iota(jnp.int32, (_SUB, _K), 1)
    z = [_proj_in(x_ref[c * _SUB:(c + 1) * _SUB], win_ref, bin_ref)
         for c in range(_NC)]
    r = list(z)
    qt = [jnp.zeros_like(z[0]) for _ in range(_NC)]
    for l in range(_L):
        cbt, cn, cbp = cbt_ref[l], cn_ref[l], cbp_ref[l]
        s = [_scores(r[c], cbt, cn) for c in range(_NC)]
        m = [jnp.min(s[c], axis=1, keepdims=True) for c in range(_NC)]
        mh = [(s[c] == m[c]).astype(jnp.float32).astype(jnp.bfloat16)
              for c in range(_NC)]
        q4 = [jnp.dot(mh[c], cbp, preferred_element_type=jnp.float32)
              for c in range(_NC)]
        nmax = q4[0][:, 3 * _DZ:3 * _DZ + 1]
        for c in range(1, _NC):
            nmax = jnp.maximum(nmax, q4[c][:, 3 * _DZ:3 * _DZ + 1])
        tied = jnp.max(nmax) > 1.5

        def _fix(s=s, m=m, cbp=cbp):
            return tuple(
                jnp.dot(_first_hit(s[c], m[c], iota), cbp,
                        preferred_element_type=jnp.float32)
                for c in range(_NC))

        q4 = jax.lax.cond(tied, _fix, lambda q4=q4: tuple(q4))
        for c in range(_NC):
            q = (q4[c][:, :_DZ] + q4[c][:, _DZ:2 * _DZ]) + \
                q4[c][:, 2 * _DZ:3 * _DZ]
            qt[c] = qt[c] + (r[c] + (q - r[c]))
            r[c] = r[c] - q
    for c in range(_NC):
        o = jnp.dot(qt[c].astype(jnp.bfloat16), wout_ref[...],
                    preferred_element_type=jnp.float32)
        out_ref[c * _SUB:(c + 1) * _SUB] = o + bout_ref[...]


def _trunc16(v):
    bits = jax.lax.bitcast_convert_type(v, jnp.uint32)
    return jax.lax.bitcast_convert_type(bits & jnp.uint32(0xFFFF0000),
                                        jnp.float32)


@functools.partial(jax.jit, static_argnames=("interpret",))
def kernel(x, proj_in_w, proj_in_b, codebooks, proj_out_w, proj_out_b,
           interpret=False):
    b, n, dims = x.shape
    tokens = b * n
    xf = x.reshape(tokens, dims).astype(jnp.bfloat16)
    cbt = jnp.swapaxes(codebooks, 1, 2).astype(jnp.bfloat16)
    hi = _trunc16(codebooks)
    rem = codebooks - hi
    mid = _trunc16(rem)
    lo = rem - mid
    ones = jnp.ones((_L, _K, 1), jnp.bfloat16)
    cbp = jnp.concatenate([hi.astype(jnp.bfloat16), mid.astype(jnp.bfloat16),
                           lo.astype(jnp.bfloat16), ones,
                           jnp.zeros((_L, _K, _DZ - 1), jnp.bfloat16)],
                          axis=-1)
    cn = jnp.sum(codebooks * codebooks, axis=-1)
    cn = cn.reshape(_L, 1, _K)
    win = proj_in_w.astype(jnp.bfloat16)
    wout = proj_out_w.astype(jnp.bfloat16)
    bin2 = proj_in_b.reshape(1, -1)
    bout2 = proj_out_b.reshape(1, -1)
    grid = (tokens // _TILE,)
    out = pl.pallas_call(
        _vq_body,
        grid=grid,
        in_specs=[
            pl.BlockSpec((_TILE, dims), lambda i: (i, 0)),
            pl.BlockSpec((dims, _DZ), lambda i: (0, 0)),
            pl.BlockSpec((1, _DZ), lambda i: (0, 0)),
            pl.BlockSpec((_L, _K, 4 * _DZ), lambda i: (0, 0, 0)),
            pl.BlockSpec((_L, _DZ, _K), lambda i: (0, 0, 0)),
            pl.BlockSpec((_L, 1, _K), lambda i: (0, 0, 0)),
            pl.BlockSpec((_DZ, dims), lambda i: (0, 0)),
            pl.BlockSpec((1, dims), lambda i: (0, 0)),
        ],
        out_specs=pl.BlockSpec((_TILE, dims), lambda i: (i, 0)),
        out_shape=jax.ShapeDtypeStruct((tokens, dims), jnp.float32),
        interpret=interpret,
    )(xf, win, bin2, cbp, cbt, cn, wout, bout2)
    return out.reshape(b, n, dims)
